# baseline (speedup 1.0000x reference)
import functools

import jax
import jax.numpy as jnp
from jax import lax
from jax.experimental import pallas as pl
from jax.experimental.pallas import tpu as pltpu
from jax.experimental.pallas import tpu_sc as plsc

VOCAB = 1000000
EMB = 64
UNITS = 128
BATCH = 1024
SEQ = 50

_NW = 32
_ROWS = BATCH * SEQ
_RPW = _ROWS // _NW
_CH = 80
_NCH = _RPW // _CH

_BB = 256


def _sc_gather(emb_table, idx3):
    mesh = plsc.VectorSubcoreMesh(core_axis_name="c", subcore_axis_name="s")

    @functools.partial(
        pl.kernel,
        mesh=mesh,
        compiler_params=pltpu.CompilerParams(use_tc_tiling_on_sc=False),
        out_type=jax.ShapeDtypeStruct((_ROWS, EMB), jnp.float32),
        scratch_types=[
            pltpu.VMEM((_NCH, _CH), jnp.int32),
            pltpu.VMEM((_RPW, EMB), jnp.float32),
            pltpu.SemaphoreType.DMA,
        ],
    )
    def gather_kernel(table_hbm, idx_hbm, out_hbm, idx_v, rows_v, sem):
        wid = lax.axis_index("s") * 2 + lax.axis_index("c")
        pltpu.sync_copy(idx_hbm.at[wid], idx_v)
        copies = [
            pltpu.async_copy(
                table_hbm.at[idx_v.at[j]],
                rows_v.at[pl.ds(j * _CH, _CH)],
                sem,
            )
            for j in range(_NCH)
        ]
        for c in copies:
            c.wait()
        pltpu.sync_copy(rows_v, out_hbm.at[pl.ds(wid * _RPW, _RPW)])

    return gather_kernel(emb_table, idx3)


def _gru_body(xe_ref, h0_ref, w_ref, rw_ref, bi_ref, br_ref, out_ref, st_ref):
    h = h0_ref[...]
    w = w_ref[...]
    rw = rw_ref[...]
    bi = bi_ref[...]
    br = br_ref[...]
    for t in range(SEQ):
        xt = xe_ref[t]
        gx = jnp.dot(xt, w, preferred_element_type=jnp.float32) + bi
        gh = jnp.dot(h, rw, preferred_element_type=jnp.float32) + br
        xz = gx[:, :UNITS]
        xr = gx[:, UNITS:2 * UNITS]
        xh = gx[:, 2 * UNITS:]
        hz = gh[:, :UNITS]
        hr = gh[:, UNITS:2 * UNITS]
        hh = gh[:, 2 * UNITS:]
        z = jax.nn.sigmoid(xz + hz)
        r = jax.nn.sigmoid(xr + hr)
        hcand = jnp.tanh(xh + r * hh)
        h = z * h + (1.0 - z) * hcand
        out_ref[t] = h
    st_ref[...] = h


def _tc_gru(xe, hidden, w, rw, bi, br):
    grid = (BATCH // _BB,)
    out, state = pl.pallas_call(
        _gru_body,
        grid=grid,
        in_specs=[
            pl.BlockSpec((SEQ, _BB, EMB), lambda i: (0, i, 0)),
            pl.BlockSpec((_BB, UNITS), lambda i: (i, 0)),
            pl.BlockSpec((EMB, 3 * UNITS), lambda i: (0, 0)),
            pl.BlockSpec((UNITS, 3 * UNITS), lambda i: (0, 0)),
            pl.BlockSpec((1, 3 * UNITS), lambda i: (0, 0)),
            pl.BlockSpec((1, 3 * UNITS), lambda i: (0, 0)),
        ],
        out_specs=[
            pl.BlockSpec((SEQ, _BB, UNITS), lambda i: (0, i, 0)),
            pl.BlockSpec((_BB, UNITS), lambda i: (i, 0)),
        ],
        out_shape=[
            jax.ShapeDtypeStruct((SEQ, BATCH, UNITS), jnp.float32),
            jax.ShapeDtypeStruct((BATCH, UNITS), jnp.float32),
        ],
    )(xe, hidden, w, rw, bi, br)
    return out, state


def kernel(x, hidden, emb_table, kernel, rec_kernel, bias_in, bias_rec):
    idx = jnp.transpose(x.astype(jnp.int32)).reshape(_NW, _NCH, _CH)
    rows = _sc_gather(emb_table, idx)
    xe = rows.reshape(SEQ, BATCH, EMB)
    bi = bias_in.reshape(1, 3 * UNITS)
    br = bias_rec.reshape(1, 3 * UNITS)
    out, state = _tc_gru(xe, hidden, kernel, rec_kernel, bi, br)
    return (jnp.swapaxes(out, 0, 1), state)
